# baseline (device time: 28749 ns/iter reference)
import jax
import jax.numpy as jnp
from jax import lax
from jax.experimental import pallas as pl
from jax.experimental.pallas import tpu as pltpu

N_DEV = 8


def kernel(x, w_mat):
    m, k_loc = x.shape
    k, n = w_mat.shape
    blk = m // N_DEV

    def body(x_hbm, w_hbm, out_hbm, x_vmem, xg_ref, w_buf, acc_ref,
             x_sem, w_sems, out_sem, send_sems, recv_sems, ready_sems):
        my = lax.axis_index("i")

        barrier_sem = pltpu.get_barrier_semaphore()
        pl.semaphore_signal(barrier_sem, inc=1)
        pl.semaphore_wait(barrier_sem, 1)

        for off in range(1, N_DEV):
            src = lax.rem(my - off + N_DEV, N_DEV)
            pl.semaphore_signal(
                ready_sems.at[off], inc=1,
                device_id=(src,), device_id_type=pl.DeviceIdType.MESH,
            )

        x_cp = pltpu.make_async_copy(x_hbm, x_vmem, x_sem)
        x_cp.start()

        def w_load(j, slot):
            src = lax.rem(my - j + N_DEV, N_DEV)
            return pltpu.make_async_copy(
                w_hbm.at[pl.ds(src * blk, blk), :], w_buf.at[slot],
                w_sems.at[slot],
            )

        w_load(0, 0).start()
        w_load(1, 1).start()
        x_cp.wait()
        xg_ref[my] = x_vmem[pl.ds(my * blk, blk), :]

        sends = []
        for off in range(1, N_DEV):
            peer = lax.rem(my + off, N_DEV)
            pl.semaphore_wait(ready_sems.at[off], 1)
            rdma = pltpu.make_async_remote_copy(
                src_ref=x_vmem.at[pl.ds(peer * blk, blk), :],
                dst_ref=xg_ref.at[my],
                send_sem=send_sems.at[off],
                recv_sem=recv_sems.at[off],
                device_id=(peer,),
                device_id_type=pl.DeviceIdType.MESH,
            )
            rdma.start()
            sends.append(rdma)

        for j in range(N_DEV):
            slot = j % 2
            w_load(j, slot).wait()
            if j >= 1:
                sends[j - 1].wait_recv()
            src = lax.rem(my - j + N_DEV, N_DEV)
            partial = jnp.dot(
                xg_ref[src], w_buf[slot], preferred_element_type=jnp.float32
            )
            if j == 0:
                acc_ref[...] = partial
            else:
                acc_ref[...] += partial
            if j + 2 < N_DEV:
                w_load(j + 2, slot).start()

        out_cp = pltpu.make_async_copy(acc_ref, out_hbm, out_sem)
        out_cp.start()
        for s in sends:
            s.wait_send()
        out_cp.wait()

    return pl.pallas_call(
        body,
        out_shape=jax.ShapeDtypeStruct((blk, n), jnp.float32),
        in_specs=[
            pl.BlockSpec(memory_space=pltpu.MemorySpace.HBM),
            pl.BlockSpec(memory_space=pltpu.MemorySpace.HBM),
        ],
        out_specs=pl.BlockSpec(memory_space=pltpu.MemorySpace.HBM),
        scratch_shapes=[
            pltpu.VMEM((m, k_loc), jnp.float32),
            pltpu.VMEM((N_DEV, blk, k_loc), jnp.float32),
            pltpu.VMEM((2, blk, n), jnp.float32),
            pltpu.VMEM((blk, n), jnp.float32),
            pltpu.SemaphoreType.DMA,
            pltpu.SemaphoreType.DMA((2,)),
            pltpu.SemaphoreType.DMA,
            pltpu.SemaphoreType.DMA((N_DEV,)),
            pltpu.SemaphoreType.DMA((N_DEV,)),
            pltpu.SemaphoreType.REGULAR((N_DEV,)),
        ],
        compiler_params=pltpu.CompilerParams(collective_id=0),
    )(x, w_mat)


# device time: 23116 ns/iter; 1.2437x vs baseline; 1.2437x over previous
import jax
import jax.numpy as jnp
from jax import lax
from jax.experimental import pallas as pl
from jax.experimental.pallas import tpu as pltpu

N_DEV = 8


def kernel(x, w_mat):
    m, k_loc = x.shape
    k, n = w_mat.shape
    blk = m // N_DEV

    x = pltpu.with_memory_space_constraint(x, pltpu.MemorySpace.HBM)
    w_mat = pltpu.with_memory_space_constraint(w_mat, pltpu.MemorySpace.HBM)

    def body(x_hbm, w_hbm, out_hbm, x_vmem, xg_ref, w_buf, acc_ref,
             x_sem, w_sems, out_sem, send_sems, recv_sems, ready_sems):
        my = lax.axis_index("i")

        barrier_sem = pltpu.get_barrier_semaphore()
        pl.semaphore_signal(barrier_sem, inc=1)
        pl.semaphore_wait(barrier_sem, 1)

        for off in range(1, N_DEV):
            src = lax.rem(my - off + N_DEV, N_DEV)
            pl.semaphore_signal(
                ready_sems.at[off], inc=1,
                device_id=(src,), device_id_type=pl.DeviceIdType.MESH,
            )

        x_cp = pltpu.make_async_copy(x_hbm, x_vmem, x_sem)
        x_cp.start()

        def w_load(j, slot):
            src = lax.rem(my - j + N_DEV, N_DEV)
            return pltpu.make_async_copy(
                w_hbm.at[pl.ds(src * blk, blk), :], w_buf.at[slot],
                w_sems.at[slot],
            )

        w_load(0, 0).start()
        w_load(1, 1).start()
        x_cp.wait()
        xg_ref[my] = x_vmem[pl.ds(my * blk, blk), :]

        sends = []
        for off in range(1, N_DEV):
            peer = lax.rem(my + off, N_DEV)
            pl.semaphore_wait(ready_sems.at[off], 1)
            rdma = pltpu.make_async_remote_copy(
                src_ref=x_vmem.at[pl.ds(peer * blk, blk), :],
                dst_ref=xg_ref.at[my],
                send_sem=send_sems.at[off],
                recv_sem=recv_sems.at[off],
                device_id=(peer,),
                device_id_type=pl.DeviceIdType.MESH,
            )
            rdma.start()
            sends.append(rdma)

        for j in range(N_DEV):
            slot = j % 2
            w_load(j, slot).wait()
            if j >= 1:
                sends[j - 1].wait_recv()
            src = lax.rem(my - j + N_DEV, N_DEV)
            partial = jnp.dot(
                xg_ref[src], w_buf[slot], preferred_element_type=jnp.float32
            )
            if j == 0:
                acc_ref[...] = partial
            else:
                acc_ref[...] += partial
            if j + 2 < N_DEV:
                w_load(j + 2, slot).start()

        out_cp = pltpu.make_async_copy(acc_ref, out_hbm, out_sem)
        out_cp.start()
        for s in sends:
            s.wait_send()
        out_cp.wait()

    return pl.pallas_call(
        body,
        out_shape=jax.ShapeDtypeStruct((blk, n), jnp.float32),
        in_specs=[
            pl.BlockSpec(memory_space=pltpu.MemorySpace.HBM),
            pl.BlockSpec(memory_space=pltpu.MemorySpace.HBM),
        ],
        out_specs=pl.BlockSpec(memory_space=pltpu.MemorySpace.HBM),
        scratch_shapes=[
            pltpu.VMEM((m, k_loc), jnp.float32),
            pltpu.VMEM((N_DEV, blk, k_loc), jnp.float32),
            pltpu.VMEM((2, blk, n), jnp.float32),
            pltpu.VMEM((blk, n), jnp.float32),
            pltpu.SemaphoreType.DMA,
            pltpu.SemaphoreType.DMA((2,)),
            pltpu.SemaphoreType.DMA,
            pltpu.SemaphoreType.DMA((N_DEV,)),
            pltpu.SemaphoreType.DMA((N_DEV,)),
            pltpu.SemaphoreType.REGULAR((N_DEV,)),
        ],
        compiler_params=pltpu.CompilerParams(collective_id=0),
    )(x, w_mat)


# device time: 22164 ns/iter; 1.2971x vs baseline; 1.0430x over previous
import jax
import jax.numpy as jnp
from jax import lax
from jax.experimental import pallas as pl
from jax.experimental.pallas import tpu as pltpu

N_DEV = 8
N_PAIR = N_DEV // 2


def kernel(x, w_mat):
    m, k_loc = x.shape
    k, n = w_mat.shape
    blk = m // N_DEV

    x = pltpu.with_memory_space_constraint(x, pltpu.MemorySpace.HBM)
    w_mat = pltpu.with_memory_space_constraint(w_mat, pltpu.MemorySpace.HBM)

    def body(x_hbm, w_hbm, out_hbm, xg_ref, w_buf, acc_ref,
             x_sem, w_sems, out_sem, send_sems, recv_sems, ready_sems):
        my = lax.axis_index("i")

        barrier_sem = pltpu.get_barrier_semaphore()
        pl.semaphore_signal(barrier_sem, inc=1)
        pl.semaphore_wait(barrier_sem, 1)

        for off in range(1, N_DEV):
            src = lax.rem(my - off + N_DEV, N_DEV)
            pl.semaphore_signal(
                ready_sems.at[off], inc=1,
                device_id=(src,), device_id_type=pl.DeviceIdType.MESH,
            )

        x_cp = pltpu.make_async_copy(
            x_hbm.at[pl.ds(my * blk, blk), :], xg_ref.at[:, pl.ds(0, blk)],
            x_sem,
        )
        x_cp.start()

        sends = []
        for off in range(1, N_DEV):
            peer = lax.rem(my + off, N_DEV)
            pl.semaphore_wait(ready_sems.at[off], 1)
            rdma = pltpu.make_async_remote_copy(
                src_ref=x_hbm.at[pl.ds(peer * blk, blk), :],
                dst_ref=xg_ref.at[:, pl.ds(off * blk, blk)],
                send_sem=send_sems.at[off],
                recv_sem=recv_sems.at[off],
                device_id=(peer,),
                device_id_type=pl.DeviceIdType.MESH,
            )
            rdma.start()
            sends.append(rdma)

        def w_load(j, slot, half):
            src = lax.rem(my - j + N_DEV, N_DEV)
            return pltpu.make_async_copy(
                w_hbm.at[pl.ds(src * blk, blk), :],
                w_buf.at[slot, pl.ds(half * blk, blk), :],
                w_sems.at[slot],
            )

        for h in (0, 1):
            w_load(h, 0, h).start()
            w_load(2 + h, 1, h).start()

        for p in range(N_PAIR):
            slot = p % 2
            for h in (0, 1):
                j = 2 * p + h
                w_load(j, slot, h).wait()
                if j == 0:
                    x_cp.wait()
                else:
                    sends[j - 1].wait_recv()
            partial = jnp.dot(
                xg_ref[:, pl.ds(2 * p * blk, 2 * blk)],
                w_buf[slot],
                preferred_element_type=jnp.float32,
            )
            if p == 0:
                acc_ref[...] = partial
            else:
                acc_ref[...] += partial
            if p + 2 < N_PAIR:
                for h in (0, 1):
                    w_load(2 * (p + 2) + h, slot, h).start()

        out_cp = pltpu.make_async_copy(acc_ref, out_hbm, out_sem)
        out_cp.start()
        for s in sends:
            s.wait_send()
        out_cp.wait()

    return pl.pallas_call(
        body,
        out_shape=jax.ShapeDtypeStruct((blk, n), jnp.float32),
        in_specs=[
            pl.BlockSpec(memory_space=pltpu.MemorySpace.HBM),
            pl.BlockSpec(memory_space=pltpu.MemorySpace.HBM),
        ],
        out_specs=pl.BlockSpec(memory_space=pltpu.MemorySpace.HBM),
        scratch_shapes=[
            pltpu.VMEM((blk, m), jnp.float32),
            pltpu.VMEM((2, 2 * blk, n), jnp.float32),
            pltpu.VMEM((blk, n), jnp.float32),
            pltpu.SemaphoreType.DMA,
            pltpu.SemaphoreType.DMA((2,)),
            pltpu.SemaphoreType.DMA,
            pltpu.SemaphoreType.DMA((N_DEV,)),
            pltpu.SemaphoreType.DMA((N_DEV,)),
            pltpu.SemaphoreType.REGULAR((N_DEV,)),
        ],
        compiler_params=pltpu.CompilerParams(collective_id=0),
    )(x, w_mat)


# device time: 21865 ns/iter; 1.3148x vs baseline; 1.0137x over previous
import jax
import jax.numpy as jnp
from jax import lax
from jax.experimental import pallas as pl
from jax.experimental.pallas import tpu as pltpu

N_DEV = 8
N_PAIR = N_DEV // 2


def kernel(x, w_mat):
    m, k_loc = x.shape
    k, n = w_mat.shape
    blk = m // N_DEV

    x = pltpu.with_memory_space_constraint(x, pltpu.MemorySpace.HBM)
    w_mat = pltpu.with_memory_space_constraint(w_mat, pltpu.MemorySpace.HBM)

    def body(x_hbm, w_hbm, out_hbm, xg_ref, w_buf, acc_ref,
             x_sem, w_sems, out_sem, send_sems, recv_sems, ready_sems):
        my = lax.axis_index("i")

        barrier_sem = pltpu.get_barrier_semaphore()
        pl.semaphore_signal(barrier_sem, inc=1)
        pl.semaphore_wait(barrier_sem, 1)

        for off in range(1, N_DEV):
            src = lax.rem(my - off + N_DEV, N_DEV)
            pl.semaphore_signal(
                ready_sems.at[off], inc=1,
                device_id=(src,), device_id_type=pl.DeviceIdType.MESH,
            )

        x_cp = pltpu.make_async_copy(
            x_hbm.at[pl.ds(my * blk, blk), :], xg_ref.at[:, pl.ds(0, blk)],
            x_sem,
        )
        x_cp.start()

        sends = []
        for off in range(1, N_DEV):
            peer = lax.rem(my + off, N_DEV)
            pl.semaphore_wait(ready_sems.at[off], 1)
            rdma = pltpu.make_async_remote_copy(
                src_ref=x_hbm.at[pl.ds(peer * blk, blk), :],
                dst_ref=xg_ref.at[:, pl.ds(off * blk, blk)],
                send_sem=send_sems.at[off],
                recv_sem=recv_sems.at[off],
                device_id=(peer,),
                device_id_type=pl.DeviceIdType.MESH,
            )
            rdma.start()
            sends.append(rdma)

        def w_load(j, slot, half):
            src = lax.rem(my - j + N_DEV, N_DEV)
            return pltpu.make_async_copy(
                w_hbm.at[pl.ds(src * blk, blk), :],
                w_buf.at[slot, pl.ds(half * blk, blk), :],
                w_sems.at[slot],
            )

        for h in (0, 1):
            w_load(h, 0, h).start()
            w_load(2 + h, 1, h).start()

        for p in range(3):
            slot = p % 2
            for h in (0, 1):
                j = 2 * p + h
                w_load(j, slot, h).wait()
                if j == 0:
                    x_cp.wait()
                else:
                    sends[j - 1].wait_recv()
            partial = jnp.dot(
                xg_ref[:, pl.ds(2 * p * blk, 2 * blk)],
                w_buf[slot],
                preferred_element_type=jnp.float32,
            )
            if p == 0:
                acc_ref[...] = partial
            else:
                acc_ref[...] += partial
            if p == 0:
                for h in (0, 1):
                    w_load(4 + h, 0, h).start()
            if p == 1:
                for h in (0, 1):
                    w_load(6 + h, 1, h).start()
        for h in (0, 1):
            j = 6 + h
            w_load(j, 1, h).wait()
            sends[j - 1].wait_recv()
            acc_ref[...] += jnp.dot(
                xg_ref[:, pl.ds(j * blk, blk)],
                w_buf[1, pl.ds(h * blk, blk), :],
                preferred_element_type=jnp.float32,
            )

        out_cp = pltpu.make_async_copy(acc_ref, out_hbm, out_sem)
        out_cp.start()
        for s in sends:
            s.wait_send()
        out_cp.wait()

    return pl.pallas_call(
        body,
        out_shape=jax.ShapeDtypeStruct((blk, n), jnp.float32),
        in_specs=[
            pl.BlockSpec(memory_space=pltpu.MemorySpace.HBM),
            pl.BlockSpec(memory_space=pltpu.MemorySpace.HBM),
        ],
        out_specs=pl.BlockSpec(memory_space=pltpu.MemorySpace.HBM),
        scratch_shapes=[
            pltpu.VMEM((blk, m), jnp.float32),
            pltpu.VMEM((2, 2 * blk, n), jnp.float32),
            pltpu.VMEM((blk, n), jnp.float32),
            pltpu.SemaphoreType.DMA,
            pltpu.SemaphoreType.DMA((2,)),
            pltpu.SemaphoreType.DMA,
            pltpu.SemaphoreType.DMA((N_DEV,)),
            pltpu.SemaphoreType.DMA((N_DEV,)),
            pltpu.SemaphoreType.REGULAR((N_DEV,)),
        ],
        compiler_params=pltpu.CompilerParams(collective_id=0),
    )(x, w_mat)


# device time: 21767 ns/iter; 1.3208x vs baseline; 1.0045x over previous
import jax
import jax.numpy as jnp
from jax import lax
from jax.experimental import pallas as pl
from jax.experimental.pallas import tpu as pltpu

N_DEV = 8
N_PAIR = N_DEV // 2

P_SEQ = (0, 1, 3, 2, 4, 5, 7, 6)


def kernel(x, w_mat):
    m, k_loc = x.shape
    k, n = w_mat.shape
    blk = m // N_DEV

    x = pltpu.with_memory_space_constraint(x, pltpu.MemorySpace.HBM)
    w_mat = pltpu.with_memory_space_constraint(w_mat, pltpu.MemorySpace.HBM)

    def body(x_hbm, w_hbm, out_hbm, xg_ref, w_buf, acc_ref,
             x_sem, w_sems, out_sem, send_sems, recv_sems, ready_sems):
        my = lax.axis_index("i")

        barrier_sem = pltpu.get_barrier_semaphore()
        pl.semaphore_signal(barrier_sem, inc=1)
        pl.semaphore_wait(barrier_sem, 1)

        for p in P_SEQ[1:]:
            pl.semaphore_signal(
                ready_sems.at[p], inc=1,
                device_id=(my ^ p,), device_id_type=pl.DeviceIdType.MESH,
            )

        x_cp = pltpu.make_async_copy(
            x_hbm.at[pl.ds(my * blk, blk), :], xg_ref.at[:, pl.ds(0, blk)],
            x_sem,
        )
        x_cp.start()

        sends = []
        for q in range(1, N_DEV):
            p = P_SEQ[q]
            peer = my ^ p
            pl.semaphore_wait(ready_sems.at[p], 1)
            rdma = pltpu.make_async_remote_copy(
                src_ref=x_hbm.at[pl.ds(peer * blk, blk), :],
                dst_ref=xg_ref.at[:, pl.ds(q * blk, blk)],
                send_sem=send_sems.at[p],
                recv_sem=recv_sems.at[p],
                device_id=(peer,),
                device_id_type=pl.DeviceIdType.MESH,
            )
            rdma.start()
            sends.append(rdma)

        def w_load(q, slot, half):
            src = my ^ P_SEQ[q]
            return pltpu.make_async_copy(
                w_hbm.at[pl.ds(src * blk, blk), :],
                w_buf.at[slot, pl.ds(half * blk, blk), :],
                w_sems.at[slot],
            )

        for h in (0, 1):
            w_load(h, 0, h).start()
            w_load(2 + h, 1, h).start()

        for p in range(3):
            slot = p % 2
            for h in (0, 1):
                j = 2 * p + h
                w_load(j, slot, h).wait()
                if j == 0:
                    x_cp.wait()
                else:
                    sends[j - 1].wait_recv()
            partial = jnp.dot(
                xg_ref[:, pl.ds(2 * p * blk, 2 * blk)],
                w_buf[slot],
                preferred_element_type=jnp.float32,
            )
            if p == 0:
                acc_ref[...] = partial
            else:
                acc_ref[...] += partial
            if p == 0:
                for h in (0, 1):
                    w_load(4 + h, 0, h).start()
            if p == 1:
                for h in (0, 1):
                    w_load(6 + h, 1, h).start()
        for h in (0, 1):
            j = 6 + h
            w_load(j, 1, h).wait()
            sends[j - 1].wait_recv()
            acc_ref[...] += jnp.dot(
                xg_ref[:, pl.ds(j * blk, blk)],
                w_buf[1, pl.ds(h * blk, blk), :],
                preferred_element_type=jnp.float32,
            )

        out_cp = pltpu.make_async_copy(acc_ref, out_hbm, out_sem)
        out_cp.start()
        for s in sends:
            s.wait_send()
        out_cp.wait()

    return pl.pallas_call(
        body,
        out_shape=jax.ShapeDtypeStruct((blk, n), jnp.float32),
        in_specs=[
            pl.BlockSpec(memory_space=pltpu.MemorySpace.HBM),
            pl.BlockSpec(memory_space=pltpu.MemorySpace.HBM),
        ],
        out_specs=pl.BlockSpec(memory_space=pltpu.MemorySpace.HBM),
        scratch_shapes=[
            pltpu.VMEM((blk, m), jnp.float32),
            pltpu.VMEM((2, 2 * blk, n), jnp.float32),
            pltpu.VMEM((blk, n), jnp.float32),
            pltpu.SemaphoreType.DMA,
            pltpu.SemaphoreType.DMA((2,)),
            pltpu.SemaphoreType.DMA,
            pltpu.SemaphoreType.DMA((N_DEV,)),
            pltpu.SemaphoreType.DMA((N_DEV,)),
            pltpu.SemaphoreType.REGULAR((N_DEV,)),
        ],
        compiler_params=pltpu.CompilerParams(collective_id=0),
    )(x, w_mat)
